# baseline (speedup 1.0000x reference)
import functools

import jax
import jax.numpy as jnp
from jax import lax
from jax.experimental import pallas as pl
from jax.experimental.pallas import tpu as pltpu
from jax.experimental.pallas import tpu_sc as plsc

B, L, D = 4096, 50, 64
V = 100000
EPS = 1e-5
N = B * L
NC, NS = 2, 16
NW = NC * NS
C = 128
ROWS_W = N // NW
NCH = ROWS_W // C
NG = N // C


def _stats_body(p_ref, w_ref, g_ref, b_ref, out_ref):
    p = p_ref[...]
    s1 = jnp.sum(p)
    s2 = jnp.sum(p * p)
    mean = s1 / N
    var = s2 / N - mean * mean
    w = w_ref[...]
    scale = w * g_ref[...] * lax.rsqrt(w * w * var + EPS)
    off = b_ref[...] - mean * scale
    out_ref[...] = jnp.concatenate([scale, off], axis=0)


def _price_affine(price2d, w, gamma, beta):
    return pl.pallas_call(
        _stats_body,
        out_shape=jax.ShapeDtypeStruct((2, D), jnp.float32),
    )(price2d, w, gamma, beta)


def _sc_body(brand, material, author, color, price, so_tbl, tf0,
             t_brand, t_material, t_author, t_color, out,
             idx_v, x_v, so_v, tf_v, g_v, o_v, sem):
    wid = lax.axis_index("s") * NC + lax.axis_index("c")
    r0 = wid * ROWS_W

    pltpu.sync_copy(brand.at[pl.ds(r0, ROWS_W)], idx_v.at[0])
    pltpu.sync_copy(material.at[pl.ds(r0, ROWS_W)], idx_v.at[1])
    pltpu.sync_copy(author.at[pl.ds(r0, ROWS_W)], idx_v.at[2])
    pltpu.sync_copy(color.at[pl.ds(r0, ROWS_W)], idx_v.at[3])
    pltpu.sync_copy(price.at[pl.ds(r0, ROWS_W)], x_v)
    pltpu.sync_copy(so_tbl, so_v)
    pltpu.sync_copy(tf0, tf_v)

    def chunk(c, _):
        cb = c * C
        tables = (t_brand, t_material, t_author, t_color)
        pltpu.async_copy(
            tables[0].at[idx_v.at[0, pl.ds(cb, C)]], g_v, sem).wait()
        for f in (1, 2, 3):
            pltpu.async_copy(
                tables[f].at[idx_v.at[f, pl.ds(cb, C)]], g_v, sem,
                add=True).wait()

        def row(j, _):
            j16 = jnp.full((16,), cb + j, jnp.int32)
            xs = plsc.load_gather(x_v, [j16])
            zs = []
            for f in range(4):
                f16 = jnp.full((16,), f, jnp.int32)
                iv = plsc.load_gather(idx_v, [f16, j16])
                zs.append(jnp.where(iv == 0, 1.0, 0.0).astype(jnp.float32))
            for blk in range(4):
                dsl = pl.ds(blk * 16, 16)
                corr = (zs[0] * tf_v[0, dsl] + zs[1] * tf_v[1, dsl]
                        + zs[2] * tf_v[2, dsl] + zs[3] * tf_v[3, dsl])
                pr = jnp.maximum(xs * so_v[0, dsl] + so_v[1, dsl], 0.0)
                o_v[j, dsl] = g_v[j, dsl] - corr + pr
            return _

        lax.fori_loop(0, C, row, None)
        pltpu.sync_copy(o_v, out.at[pl.ds(r0 + cb, C)])
        return _

    lax.fori_loop(0, NCH, chunk, None)


_sc_call = functools.partial(
    pl.kernel,
    out_type=jax.ShapeDtypeStruct((N, D), jnp.float32),
    mesh=plsc.VectorSubcoreMesh(core_axis_name="c", subcore_axis_name="s"),
    compiler_params=pltpu.CompilerParams(
        needs_layout_passes=False, use_tc_tiling_on_sc=False),
    scratch_types=[
        pltpu.VMEM((4, ROWS_W), jnp.int32),
        pltpu.VMEM((ROWS_W,), jnp.float32),
        pltpu.VMEM((2, D), jnp.float32),
        pltpu.VMEM((4, D), jnp.float32),
        pltpu.VMEM((C, D), jnp.float32),
        pltpu.VMEM((C, D), jnp.float32),
        pltpu.SemaphoreType.DMA,
    ],
)


def kernel(brand, material, author, color, price, W_price, bn_gamma, bn_beta,
           brand_table, material_table, author_table, color_table):
    so_tbl = _price_affine(price.reshape(NG, C), W_price,
                           bn_gamma.reshape(1, D), bn_beta.reshape(1, D))
    tf0 = jnp.stack([brand_table[0], material_table[0],
                     author_table[0], color_table[0]])
    sc = _sc_call(_sc_body)
    out = sc(brand.reshape(N), material.reshape(N),
             author.reshape(N), color.reshape(N),
             price.reshape(N), so_tbl, tf0,
             brand_table, material_table, author_table, color_table)
    return out.reshape(B, L, D)


# speedup vs baseline: 1.7827x; 1.7827x over previous
import functools

import jax
import jax.numpy as jnp
from jax import lax
from jax.experimental import pallas as pl
from jax.experimental.pallas import tpu as pltpu
from jax.experimental.pallas import tpu_sc as plsc

B, L, D = 4096, 50, 64
V = 100000
EPS = 1e-5
N = B * L
NC, NS = 2, 16
NW = NC * NS
C = 128
ROWS_W = N // NW
NCH = ROWS_W // C
NG = N // C


def _stats_body(p_ref, w_ref, g_ref, b_ref, out_ref):
    p = p_ref[...]
    s1 = jnp.sum(p)
    s2 = jnp.sum(p * p)
    mean = s1 / N
    var = s2 / N - mean * mean
    w = w_ref[...]
    scale = w * g_ref[...] * lax.rsqrt(w * w * var + EPS)
    off = b_ref[...] - mean * scale
    out_ref[...] = jnp.concatenate([scale, off], axis=0)


def _price_affine(price2d, w, gamma, beta):
    return pl.pallas_call(
        _stats_body,
        out_shape=jax.ShapeDtypeStruct((2, D), jnp.float32),
    )(price2d, w, gamma, beta)


def _sc_body(brand, material, author, color, price, so_tbl, tf0,
             t_brand, t_material, t_author, t_color, out,
             idx_v, x_v, so_v, tf_v, acc0, acc1, g0, g1, o0, o1):
    wid = lax.axis_index("s") * NC + lax.axis_index("c")
    r0 = wid * ROWS_W
    tables = (t_brand, t_material, t_author, t_color)
    accs = (acc0, acc1)
    gsems = (g0, g1)
    osems = (o0, o1)

    pltpu.sync_copy(brand.at[pl.ds(r0, ROWS_W)], idx_v.at[0])
    pltpu.sync_copy(material.at[pl.ds(r0, ROWS_W)], idx_v.at[1])
    pltpu.sync_copy(author.at[pl.ds(r0, ROWS_W)], idx_v.at[2])
    pltpu.sync_copy(color.at[pl.ds(r0, ROWS_W)], idx_v.at[3])
    pltpu.sync_copy(price.at[pl.ds(r0, ROWS_W)], x_v)
    pltpu.sync_copy(so_tbl, so_v)
    pltpu.sync_copy(tf0, tf_v)

    sob = [(so_v[0, pl.ds(b * 16, 16)], so_v[1, pl.ds(b * 16, 16)])
           for b in range(4)]

    def price_fill(c, acc):
        cb = c * C

        def row(j, carry):
            j16 = jnp.full((16,), cb + j, jnp.int32)
            xs = plsc.load_gather(x_v, [j16])
            for blk in range(4):
                s, o = carry[2 * blk], carry[2 * blk + 1]
                acc[j, pl.ds(blk * 16, 16)] = jnp.maximum(xs * s + o, 0.0)
            return carry

        carry0 = tuple(v for pair in sob for v in pair)
        lax.fori_loop(0, C, row, carry0)

    def fire_gathers(c, slot):
        cb = c * C
        return [pltpu.async_copy(
            tables[f].at[idx_v.at[f, pl.ds(cb, C)]], accs[slot],
            gsems[slot], add=True) for f in range(4)]

    def wait_gathers(c, slot):
        cb = c * C
        for f in range(4):
            pltpu.make_async_copy(
                tables[f].at[idx_v.at[f, pl.ds(cb, C)]], accs[slot],
                gsems[slot]).wait()

    def fixup(c, acc):
        cb = c * C

        def grp(g, _):
            gb = cb + g * 16
            iv = [idx_v[f, pl.ds(gb, 16)] for f in range(4)]
            bad = ((iv[0] == 0) | (iv[1] == 0) | (iv[2] == 0)
                   | (iv[3] == 0))

            def dofix():
                def rr(j, _):
                    j16 = jnp.full((16,), gb + j, jnp.int32)
                    zs = []
                    for f in range(4):
                        f16 = jnp.full((16,), f, jnp.int32)
                        zf = plsc.load_gather(idx_v, [f16, j16])
                        zs.append(jnp.where(zf == 0, -1.0, 0.0))
                    lr = g * 16 + j
                    for blk in range(4):
                        dsl = pl.ds(blk * 16, 16)
                        corr = (zs[0] * tf_v[0, dsl] + zs[1] * tf_v[1, dsl]
                                + zs[2] * tf_v[2, dsl] + zs[3] * tf_v[3, dsl])
                        plsc.addupdate(acc.at[lr, dsl], corr)
                    return _

                lax.fori_loop(0, 16, rr, None)

            lax.cond(jnp.any(bad), dofix, lambda: None)
            return _

        lax.fori_loop(0, C // 16, grp, None)

    def fire_out(c, slot):
        cb = c * C
        return pltpu.async_copy(accs[slot], out.at[pl.ds(r0 + cb, C)],
                                osems[slot])

    def wait_out(c, slot):
        cb = c * C
        pltpu.make_async_copy(accs[slot], out.at[pl.ds(r0 + cb, C)],
                              osems[slot]).wait()

    price_fill(0, accs[0])
    fire_gathers(0, 0)

    def S(c, p, q):
        @pl.when(c + 1 < NCH)
        def _issue():
            @pl.when(c >= 1)
            def _w():
                wait_out(c - 1, p)
            price_fill(c + 1, accs[p])
            fire_gathers(c + 1, p)

        wait_gathers(c, q)
        fixup(c, accs[q])
        fire_out(c, q)

    def pair(i, _):
        c = 2 * i
        S(c, 1, 0)
        S(c + 1, 0, 1)
        return _

    lax.fori_loop(0, NCH // 2, pair, None)

    wait_out(NCH - 2, 0)
    wait_out(NCH - 1, 1)


_sc_call = functools.partial(
    pl.kernel,
    out_type=jax.ShapeDtypeStruct((N, D), jnp.float32),
    mesh=plsc.VectorSubcoreMesh(core_axis_name="c", subcore_axis_name="s"),
    compiler_params=pltpu.CompilerParams(
        needs_layout_passes=False, use_tc_tiling_on_sc=False),
    scratch_types=[
        pltpu.VMEM((4, ROWS_W), jnp.int32),
        pltpu.VMEM((ROWS_W,), jnp.float32),
        pltpu.VMEM((2, D), jnp.float32),
        pltpu.VMEM((4, D), jnp.float32),
        pltpu.VMEM((C, D), jnp.float32),
        pltpu.VMEM((C, D), jnp.float32),
        pltpu.SemaphoreType.DMA,
        pltpu.SemaphoreType.DMA,
        pltpu.SemaphoreType.DMA,
        pltpu.SemaphoreType.DMA,
    ],
)


def kernel(brand, material, author, color, price, W_price, bn_gamma, bn_beta,
           brand_table, material_table, author_table, color_table):
    so_tbl = _price_affine(price.reshape(NG, C), W_price,
                           bn_gamma.reshape(1, D), bn_beta.reshape(1, D))
    tf0 = jnp.stack([brand_table[0], material_table[0],
                     author_table[0], color_table[0]])
    sc = _sc_call(_sc_body)
    out = sc(brand.reshape(N), material.reshape(N),
             author.reshape(N), color.reshape(N),
             price.reshape(N), so_tbl, tf0,
             brand_table, material_table, author_table, color_table)
    return out.reshape(B, L, D)
